# Initial kernel scaffold; baseline (speedup 1.0000x reference)
import jax
import jax.numpy as jnp
from jax.experimental import pallas as pl


def kernel(x, edge_index, W1, b1, W2, b2):
    raise NotImplementedError("write your pallas kernel here")


# speedup vs baseline: 6.0193x; 6.0193x over previous
import functools

import jax
import jax.numpy as jnp
from jax import lax
from jax.experimental import pallas as pl
from jax.experimental.pallas import tpu as pltpu
from jax.experimental.pallas import tpu_sc as plsc

N = 10000
E = 160000
D = 256
NC, NS = 2, 16
NPAD = 10240
TRASH = N
CW = 128
CH = 80
E_PAD = NS * CH * CW
RPT = NPAD // NS
DEG_CH = CH // NC
DH = D // NC

_mesh = plsc.VectorSubcoreMesh(
    core_axis_name="c", subcore_axis_name="s", num_cores=NC, num_subcores=NS)


def _deg_body(dst_hbm, ones_hbm, zeros_hbm, out_hbm, dst_v, ones_v, acc_sh):
    c = lax.axis_index("c")
    s = lax.axis_index("s")
    pltpu.sync_copy(zeros_hbm.at[pl.ds(0, RPT)], acc_sh.at[pl.ds(s * RPT, RPT)])
    pltpu.sync_copy(ones_hbm, ones_v)
    pltpu.sync_copy(dst_hbm.at[c, s], dst_v)
    plsc.subcore_barrier()

    def body(j, carry):
        pltpu.sync_copy(ones_v, acc_sh.at[dst_v.at[j]], add=True)
        return carry

    lax.fori_loop(0, DEG_CH, body, 0)
    plsc.subcore_barrier()
    pltpu.sync_copy(acc_sh.at[pl.ds(s * RPT, RPT)],
                    out_hbm.at[c, pl.ds(s * RPT, RPT)])


def _make_deg_kernel(interpret=False, width=DH):
    return functools.partial(
        pl.kernel,
        out_type=jax.ShapeDtypeStruct((NC, NPAD, width), jnp.float32),
        mesh=_mesh,
        scratch_types=[
            pltpu.VMEM((DEG_CH, CW), jnp.int32),
            pltpu.VMEM((CW, width), jnp.float32),
            pltpu.VMEM_SHARED((NPAD, width), jnp.float32),
        ],
        interpret=interpret,
    )(_deg_body)


def _agg_body(g_hbm, src_hbm, dst_hbm, zeros_hbm, out_hbm,
              src_v, dst_v, gbuf, acc_sh, sem):
    c = lax.axis_index("c")
    s = lax.axis_index("s")
    pltpu.sync_copy(zeros_hbm.at[pl.ds(0, RPT)], acc_sh.at[pl.ds(s * RPT, RPT)])
    pltpu.sync_copy(src_hbm.at[c, s], src_v)
    pltpu.sync_copy(dst_hbm.at[s], dst_v)
    plsc.subcore_barrier()

    def body(j, carry):
        pltpu.async_copy(g_hbm.at[src_v.at[j]], gbuf, sem).wait()
        pltpu.sync_copy(gbuf, acc_sh.at[dst_v.at[j]], add=True)
        return carry

    lax.fori_loop(0, CH, body, 0)
    plsc.subcore_barrier()
    pltpu.sync_copy(acc_sh.at[pl.ds(s * RPT, RPT)],
                    out_hbm.at[c, pl.ds(s * RPT, RPT)])


def _make_agg_kernel(interpret=False):
    return functools.partial(
        pl.kernel,
        out_type=jax.ShapeDtypeStruct((NC, NPAD, DH), jnp.float32),
        mesh=_mesh,
        scratch_types=[
            pltpu.VMEM((CH, CW), jnp.int32),
            pltpu.VMEM((CH, CW), jnp.int32),
            pltpu.VMEM((CW, DH), jnp.float32),
            pltpu.VMEM_SHARED((NPAD, DH), jnp.float32),
            pltpu.SemaphoreType.DMA,
        ],
        interpret=interpret,
    )(_agg_body)


_deg_kernel = _make_deg_kernel()
_agg_kernel = _make_agg_kernel()


def _dinv_from(deg_ref):
    dt = deg_ref[0, :, 0:1] + deg_ref[1, :, 0:1] + 1.0
    return lax.rsqrt(dt)


def _pre_body(x_ref, w_ref, deg_ref, out_ref):
    dinv = _dinv_from(deg_ref)
    h = jnp.dot(x_ref[...], w_ref[...], preferred_element_type=jnp.float32)
    out_ref[0] = dinv * h


def _mid_body(acc_ref, g_ref, deg_ref, b_ref, w_ref, out_ref):
    dinv = _dinv_from(deg_ref)
    u0 = dinv * (acc_ref[0] + g_ref[0]) + b_ref[0:1, :]
    u1 = dinv * (acc_ref[1] + g_ref[1]) + b_ref[1:2, :]
    u0 = jnp.where(u0 >= 0, u0, 0.01 * u0)
    u1 = jnp.where(u1 >= 0, u1, 0.01 * u1)
    h = (jnp.dot(u0, w_ref[0:DH, :], preferred_element_type=jnp.float32)
         + jnp.dot(u1, w_ref[DH:, :], preferred_element_type=jnp.float32))
    out_ref[0] = dinv * h


def _fin_body(acc_ref, g_ref, deg_ref, b_ref, out_ref):
    dinv = _dinv_from(deg_ref)
    out_ref[:, 0:DH] = dinv * (acc_ref[0] + g_ref[0]) + b_ref[0:1, :]
    out_ref[:, DH:] = dinv * (acc_ref[1] + g_ref[1]) + b_ref[1:2, :]


_BM = 256
_NB = NPAD // _BM

_pre_call = pl.pallas_call(
    _pre_body,
    grid=(_NB, NC),
    in_specs=[
        pl.BlockSpec((_BM, D), lambda i, j: (i, 0)),
        pl.BlockSpec((D, DH), lambda i, j: (0, j)),
        pl.BlockSpec((NC, _BM, DH), lambda i, j: (0, i, 0)),
    ],
    out_specs=pl.BlockSpec((1, _BM, DH), lambda i, j: (j, i, 0)),
    out_shape=jax.ShapeDtypeStruct((NC, NPAD, DH), jnp.float32),
)

_mid_call = pl.pallas_call(
    _mid_body,
    grid=(_NB, NC),
    in_specs=[
        pl.BlockSpec((NC, _BM, DH), lambda i, j: (0, i, 0)),
        pl.BlockSpec((NC, _BM, DH), lambda i, j: (0, i, 0)),
        pl.BlockSpec((NC, _BM, DH), lambda i, j: (0, i, 0)),
        pl.BlockSpec((NC, DH), lambda i, j: (0, 0)),
        pl.BlockSpec((D, DH), lambda i, j: (0, j)),
    ],
    out_specs=pl.BlockSpec((1, _BM, DH), lambda i, j: (j, i, 0)),
    out_shape=jax.ShapeDtypeStruct((NC, NPAD, DH), jnp.float32),
)

_fin_call = pl.pallas_call(
    _fin_body,
    grid=(_NB,),
    in_specs=[
        pl.BlockSpec((NC, _BM, DH), lambda i: (0, i, 0)),
        pl.BlockSpec((NC, _BM, DH), lambda i: (0, i, 0)),
        pl.BlockSpec((NC, _BM, DH), lambda i: (0, i, 0)),
        pl.BlockSpec((NC, DH), lambda i: (0, 0)),
    ],
    out_specs=pl.BlockSpec((_BM, D), lambda i: (i, 0)),
    out_shape=jax.ShapeDtypeStruct((NPAD, D), jnp.float32),
)


def kernel(x, edge_index, W1, b1, W2, b2):
    src = edge_index[0].astype(jnp.int32)
    dst = edge_index[1].astype(jnp.int32)
    pad = jnp.full((E_PAD - E,), TRASH, jnp.int32)
    src_t = jnp.concatenate([src, pad]).reshape(NS, CH, CW)
    dst_t = jnp.concatenate([dst, pad]).reshape(NS, CH, CW)
    src2 = jnp.stack([src_t, src_t + NPAD])

    x_pad = jnp.zeros((NPAD, D), jnp.float32).at[:N].set(x)
    zeros_wide = jnp.zeros((RPT, DH), jnp.float32)
    ones_deg = jnp.ones((CW, DH), jnp.float32)

    dst4 = jnp.stack([dst_t[:, :DEG_CH], dst_t[:, DEG_CH:]])
    deg = _deg_kernel(dst4, ones_deg, zeros_wide)
    g1 = _pre_call(x_pad, W1, deg)
    acc1 = _agg_kernel(g1.reshape(NC * NPAD, DH), src2, dst_t, zeros_wide)
    g2 = _mid_call(acc1, g1, deg, b1.reshape(NC, DH), W2)
    acc2 = _agg_kernel(g2.reshape(NC * NPAD, DH), src2, dst_t, zeros_wide)
    out = _fin_call(acc2, g2, deg, b2.reshape(NC, DH))
    return out[:N]


# speedup vs baseline: 6.9908x; 1.1614x over previous
import functools

import jax
import jax.numpy as jnp
from jax import lax
from jax.experimental import pallas as pl
from jax.experimental.pallas import tpu as pltpu
from jax.experimental.pallas import tpu_sc as plsc

N = 10000
E = 160000
D = 256
NC, NS = 2, 16
NPAD = 10240
TRASH = N
CW = 128
CH = 80
E_PAD = NS * CH * CW
RPT = NPAD // NS
DEG_CH = CH // NC
DH = D // NC

_mesh = plsc.VectorSubcoreMesh(
    core_axis_name="c", subcore_axis_name="s", num_cores=NC, num_subcores=NS)


def _deg_body(dst_hbm, ones_hbm, zeros_hbm, out_hbm, dst_v, ones_v, acc_sh):
    c = lax.axis_index("c")
    s = lax.axis_index("s")
    pltpu.sync_copy(zeros_hbm.at[pl.ds(0, RPT)], acc_sh.at[pl.ds(s * RPT, RPT)])
    pltpu.sync_copy(ones_hbm, ones_v)
    pltpu.sync_copy(dst_hbm.at[c, s], dst_v)
    plsc.subcore_barrier()

    def body(j, carry):
        pltpu.sync_copy(ones_v, acc_sh.at[dst_v.at[j]], add=True)
        return carry

    lax.fori_loop(0, DEG_CH, body, 0)
    plsc.subcore_barrier()
    pltpu.sync_copy(acc_sh.at[pl.ds(s * RPT, RPT)],
                    out_hbm.at[c, pl.ds(s * RPT, RPT)])


def _make_deg_kernel(interpret=False, width=DH):
    return functools.partial(
        pl.kernel,
        out_type=jax.ShapeDtypeStruct((NC, NPAD, width), jnp.float32),
        mesh=_mesh,
        scratch_types=[
            pltpu.VMEM((DEG_CH, CW), jnp.int32),
            pltpu.VMEM((CW, width), jnp.float32),
            pltpu.VMEM_SHARED((NPAD, width), jnp.float32),
        ],
        interpret=interpret,
    )(_deg_body)


NBUF = 2
ISL = 4


def _agg_body(g_hbm, src_hbm, dst_hbm, zeros_hbm, out_hbm,
              sidx, dst_v, gbuf, acc_sh, sem_g, sem_i):
    c = lax.axis_index("c")
    s = lax.axis_index("s")
    pltpu.sync_copy(zeros_hbm.at[pl.ds(0, RPT)], acc_sh.at[pl.ds(s * RPT, RPT)])
    pltpu.sync_copy(dst_hbm.at[s], dst_v)
    for t in range(ISL):
        pltpu.async_copy(src_hbm.at[c, s, t], sidx.at[t], sem_i)
    plsc.subcore_barrier()
    for b in range(NBUF):
        pltpu.make_async_copy(src_hbm.at[c, s, b], sidx.at[b], sem_i).wait()
        pltpu.async_copy(g_hbm.at[sidx.at[b]], gbuf.at[b], sem_g)

    def body(j, carry):
        b = lax.rem(j, NBUF)
        pltpu.make_async_copy(g_hbm.at[sidx.at[b]], gbuf.at[b], sem_g).wait()
        pltpu.sync_copy(gbuf.at[b], acc_sh.at[dst_v.at[j]], add=True)
        jn = j + NBUF

        @pl.when(jn < CH)
        def _():
            sl = lax.rem(jn, ISL)
            pltpu.make_async_copy(src_hbm.at[c, s, jn], sidx.at[sl],
                                  sem_i).wait()
            pltpu.async_copy(g_hbm.at[sidx.at[sl]], gbuf.at[b], sem_g)
            jp = jn + (ISL - NBUF)

            @pl.when(jp < CH)
            def _():
                pltpu.async_copy(src_hbm.at[c, s, jp],
                                 sidx.at[lax.rem(jp, ISL)], sem_i)

        return carry

    lax.fori_loop(0, CH, body, 0)
    plsc.subcore_barrier()
    pltpu.sync_copy(acc_sh.at[pl.ds(s * RPT, RPT)],
                    out_hbm.at[c, pl.ds(s * RPT, RPT)])


def _make_agg_kernel(interpret=False):
    return functools.partial(
        pl.kernel,
        out_type=jax.ShapeDtypeStruct((NC, NPAD, DH), jnp.float32),
        mesh=_mesh,
        scratch_types=[
            pltpu.VMEM((ISL, CW), jnp.int32),
            pltpu.VMEM((CH, CW), jnp.int32),
            pltpu.VMEM((NBUF, CW, DH), jnp.float32),
            pltpu.VMEM_SHARED((NPAD, DH), jnp.float32),
            pltpu.SemaphoreType.DMA,
            pltpu.SemaphoreType.DMA,
        ],
        interpret=interpret,
    )(_agg_body)


_deg_kernel = _make_deg_kernel()
_agg_kernel = _make_agg_kernel()


def _dinv_from(deg_ref):
    dt = deg_ref[0, :, 0:1] + deg_ref[1, :, 0:1] + 1.0
    return lax.rsqrt(dt)


def _pre_body(x_ref, w_ref, deg_ref, out_ref):
    dinv = _dinv_from(deg_ref)
    h = jnp.dot(x_ref[...], w_ref[...], preferred_element_type=jnp.float32)
    out_ref[0] = dinv * h


def _mid_body(acc_ref, g_ref, deg_ref, b_ref, w_ref, out_ref):
    dinv = _dinv_from(deg_ref)
    u0 = dinv * (acc_ref[0] + g_ref[0]) + b_ref[0:1, :]
    u1 = dinv * (acc_ref[1] + g_ref[1]) + b_ref[1:2, :]
    u0 = jnp.where(u0 >= 0, u0, 0.01 * u0)
    u1 = jnp.where(u1 >= 0, u1, 0.01 * u1)
    h = (jnp.dot(u0, w_ref[0:DH, :], preferred_element_type=jnp.float32)
         + jnp.dot(u1, w_ref[DH:, :], preferred_element_type=jnp.float32))
    out_ref[0] = dinv * h


def _fin_body(acc_ref, g_ref, deg_ref, b_ref, out_ref):
    dinv = _dinv_from(deg_ref)
    out_ref[:, 0:DH] = dinv * (acc_ref[0] + g_ref[0]) + b_ref[0:1, :]
    out_ref[:, DH:] = dinv * (acc_ref[1] + g_ref[1]) + b_ref[1:2, :]


_BM = 256
_NB = NPAD // _BM

_pre_call = pl.pallas_call(
    _pre_body,
    grid=(_NB, NC),
    in_specs=[
        pl.BlockSpec((_BM, D), lambda i, j: (i, 0)),
        pl.BlockSpec((D, DH), lambda i, j: (0, j)),
        pl.BlockSpec((NC, _BM, DH), lambda i, j: (0, i, 0)),
    ],
    out_specs=pl.BlockSpec((1, _BM, DH), lambda i, j: (j, i, 0)),
    out_shape=jax.ShapeDtypeStruct((NC, NPAD, DH), jnp.float32),
)

_mid_call = pl.pallas_call(
    _mid_body,
    grid=(_NB, NC),
    in_specs=[
        pl.BlockSpec((NC, _BM, DH), lambda i, j: (0, i, 0)),
        pl.BlockSpec((NC, _BM, DH), lambda i, j: (0, i, 0)),
        pl.BlockSpec((NC, _BM, DH), lambda i, j: (0, i, 0)),
        pl.BlockSpec((NC, DH), lambda i, j: (0, 0)),
        pl.BlockSpec((D, DH), lambda i, j: (0, j)),
    ],
    out_specs=pl.BlockSpec((1, _BM, DH), lambda i, j: (j, i, 0)),
    out_shape=jax.ShapeDtypeStruct((NC, NPAD, DH), jnp.float32),
)

_fin_call = pl.pallas_call(
    _fin_body,
    grid=(_NB,),
    in_specs=[
        pl.BlockSpec((NC, _BM, DH), lambda i: (0, i, 0)),
        pl.BlockSpec((NC, _BM, DH), lambda i: (0, i, 0)),
        pl.BlockSpec((NC, _BM, DH), lambda i: (0, i, 0)),
        pl.BlockSpec((NC, DH), lambda i: (0, 0)),
    ],
    out_specs=pl.BlockSpec((_BM, D), lambda i: (i, 0)),
    out_shape=jax.ShapeDtypeStruct((NPAD, D), jnp.float32),
)


def kernel(x, edge_index, W1, b1, W2, b2):
    src = edge_index[0].astype(jnp.int32)
    dst = edge_index[1].astype(jnp.int32)
    pad = jnp.full((E_PAD - E,), TRASH, jnp.int32)
    src_t = jnp.concatenate([src, pad]).reshape(NS, CH, CW)
    dst_t = jnp.concatenate([dst, pad]).reshape(NS, CH, CW)
    src2 = jnp.stack([src_t, src_t + NPAD])

    x_pad = jnp.zeros((NPAD, D), jnp.float32).at[:N].set(x)
    zeros_wide = jnp.zeros((RPT, DH), jnp.float32)
    ones_deg = jnp.ones((CW, DH), jnp.float32)

    dst4 = jnp.stack([dst_t[:, :DEG_CH], dst_t[:, DEG_CH:]])
    deg = _deg_kernel(dst4, ones_deg, zeros_wide)
    g1 = _pre_call(x_pad, W1, deg)
    acc1 = _agg_kernel(g1.reshape(NC * NPAD, DH), src2, dst_t, zeros_wide)
    g2 = _mid_call(acc1, g1, deg, b1.reshape(NC, DH), W2)
    acc2 = _agg_kernel(g2.reshape(NC * NPAD, DH), src2, dst_t, zeros_wide)
    out = _fin_call(acc2, g2, deg, b2.reshape(NC, DH))
    return out[:N]
